# baseline (speedup 1.0000x reference)
import functools
import math

import jax
import jax.numpy as jnp
from jax import lax
from jax.experimental import pallas as pl
from jax.experimental.pallas import tpu as pltpu
from jax.experimental.pallas import tpu_sc as plsc

D = 32
L = 16
NC, NS = 2, 16
NW = NC * NS

B = 4096 * 200
B_PER_W = B // NW
NBUF = 2
CHUNK = 512
N_CHUNKS = B_PER_W // CHUNK
BLK = 128
NBLK = CHUNK // BLK

_SCALE = math.sqrt(float(D))


def _body(x_hbm, table_hbm, o6_hbm, idx_v, rows, stages, gsems, ssems):
    wid = lax.axis_index("s") * NC + lax.axis_index("c")
    base = wid * B_PER_W
    pltpu.sync_copy(x_hbm.at[pl.ds(base, B_PER_W)], idx_v)
    iota16 = lax.iota(jnp.int32, L)

    def gather_start(c, b):
        pltpu.async_copy(
            table_hbm.at[idx_v.at[pl.ds(c * CHUNK, CHUNK)]], rows[b], gsems[b])

    def gather_wait(c, b):
        pltpu.make_async_copy(
            table_hbm.at[idx_v.at[pl.ds(c * CHUNK, CHUNK)]], rows[b],
            gsems[b]).wait()

    def store_start(beta, k):
        s = beta >> 5
        b4t = beta & 31
        pltpu.async_copy(
            stages[k].at[:, :, pl.ds(0, BLK)], o6_hbm.at[s, :, b4t, :, :],
            ssems[k])

    def store_wait(beta, k):
        s = beta >> 5
        b4t = beta & 31
        pltpu.make_async_copy(
            stages[k].at[:, :, pl.ds(0, BLK)], o6_hbm.at[s, :, b4t, :, :],
            ssems[k]).wait()

    dt_lo, di_lo = iota16 >> 3, iota16 & 7
    dt_hi, di_hi = dt_lo + 2, di_lo

    def build_all(rows_ref):
        def per_row(i, carry):
            col = jnp.zeros((L,), jnp.int32) + i
            for k in range(NBLK):
                r = k * BLK + i
                v0 = rows_ref[r, pl.ds(0, L)] * _SCALE
                v1 = rows_ref[r, pl.ds(L, L)] * _SCALE
                plsc.store_scatter(stages[k], [dt_lo, di_lo, col], v0)
                plsc.store_scatter(stages[k], [dt_hi, di_hi, col], v1)
            return carry

        lax.fori_loop(0, BLK, per_row, 0)

    for b in range(NBUF):
        gather_start(b, b)

    def chunk_body(c, b):
        gather_wait(c, b)
        beta0 = wid * (N_CHUNKS * NBLK) + c * NBLK

        def drain(k):
            if b == 0:
                @pl.when(c > 0)
                def _():
                    store_wait(beta0 + k, k)
            else:
                store_wait(beta0 + k, k)

        for k in range(NBLK):
            drain(k)
        build_all(rows[b])
        for k in range(NBLK):
            store_start(beta0 + k, k)

        @pl.when(c + NBUF < N_CHUNKS)
        def _():
            gather_start(c + NBUF, b)

    def outer(g, carry):
        for b in range(NBUF):
            chunk_body(g + b, b)
        return carry

    lax.fori_loop(0, N_CHUNKS // NBUF, lambda i, cr: outer(i * NBUF, cr), 0)
    for k in range(NBLK):
        beta = wid * (N_CHUNKS * NBLK) + (N_CHUNKS - 1) * NBLK + k
        store_wait(beta, k)


@functools.partial(
    pl.kernel,
    mesh=plsc.VectorSubcoreMesh(core_axis_name="c", subcore_axis_name="s"),
    out_type=jax.ShapeDtypeStruct((200, 4, 32, 8, 128), jnp.float32),
    scratch_types=[
        pltpu.VMEM((B_PER_W,), jnp.int32),
        [pltpu.VMEM((CHUNK, D), jnp.float32) for _ in range(NBUF)],
        [pltpu.VMEM((4, 8, 129), jnp.float32) for _ in range(NBLK)],
        [pltpu.SemaphoreType.DMA for _ in range(NBUF)],
        [pltpu.SemaphoreType.DMA for _ in range(NBLK)],
    ],
    compiler_params=pltpu.CompilerParams(
        use_tc_tiling_on_sc=False, needs_layout_passes=False),
)
def _gather_scale(x_hbm, table_hbm, o6_hbm, idx_v, rows, stages, gsems, ssems):
    _body(x_hbm, table_hbm, o6_hbm, idx_v, rows, stages, gsems, ssems)


def kernel(x, table):
    xt = x.T.reshape(B)
    o6 = _gather_scale(xt, table)
    return o6.transpose(2, 4, 0, 1, 3).reshape(4096, 200, 32)


# speedup vs baseline: 1.6006x; 1.6006x over previous
import functools
import math

import jax
import jax.numpy as jnp
from jax import lax
from jax.experimental import pallas as pl
from jax.experimental.pallas import tpu as pltpu
from jax.experimental.pallas import tpu_sc as plsc

D = 32
L = 16
NC, NS = 2, 16
NW = NC * NS

B = 4096 * 200
B_PER_W = B // NW
NBUF = 2
CHUNK = 512
N_CHUNKS = B_PER_W // CHUNK
BLK = 128
NBLK = CHUNK // BLK

_SCALE = math.sqrt(float(D))


def _body(x_hbm, table_hbm, o6_hbm, idx_v, rows, stages, gsems, ssems):
    wid = lax.axis_index("s") * NC + lax.axis_index("c")
    base = wid * B_PER_W
    pltpu.sync_copy(x_hbm.at[pl.ds(base, B_PER_W)], idx_v)
    iota16 = lax.iota(jnp.int32, L)

    def fix_idx(q, carry):
        for u in range(4):
            sl = pl.ds((q * 4 + u) * L, L)
            t = idx_v[sl]
            idx_v[sl] = ((t & ~32767) | ((t & 8191) << 2)
                         | ((t >> 13) & 3))
        return carry

    lax.fori_loop(0, B_PER_W // (4 * L), fix_idx, 0)

    def gather_start(c, b):
        pltpu.async_copy(
            table_hbm.at[idx_v.at[pl.ds(c * CHUNK, CHUNK)]], rows[b], gsems[b])

    def gather_wait(c, b):
        pltpu.make_async_copy(
            table_hbm.at[idx_v.at[pl.ds(c * CHUNK, CHUNK)]], rows[b],
            gsems[b]).wait()

    def store_start(beta, k):
        s = beta >> 5
        b4t = beta & 31
        pltpu.async_copy(
            stages[k].at[:, :, pl.ds(0, BLK)], o6_hbm.at[s, :, b4t, :, :],
            ssems[k])

    def store_wait(beta, k):
        s = beta >> 5
        b4t = beta & 31
        pltpu.make_async_copy(
            stages[k].at[:, :, pl.ds(0, BLK)], o6_hbm.at[s, :, b4t, :, :],
            ssems[k]).wait()

    dt_lo, di_lo = iota16 >> 3, iota16 & 7
    dt_hi, di_hi = dt_lo + 2, di_lo

    def build_all(rows_ref):
        def per_row(i, carry):
            col = jnp.zeros((L,), jnp.int32) + i
            for k in range(NBLK):
                r = k * BLK + i
                v0 = rows_ref[r, pl.ds(0, L)]
                v1 = rows_ref[r, pl.ds(L, L)]
                plsc.store_scatter(stages[k], [dt_lo, di_lo, col], v0)
                plsc.store_scatter(stages[k], [dt_hi, di_hi, col], v1)
            return carry

        lax.fori_loop(0, BLK, per_row, 0)

    for b in range(NBUF):
        gather_start(b, b)

    def chunk_body(c, b):
        gather_wait(c, b)
        beta0 = wid * (N_CHUNKS * NBLK) + c * NBLK

        def drain(k):
            if b == 0:
                @pl.when(c > 0)
                def _():
                    store_wait(beta0 + k, k)
            else:
                store_wait(beta0 + k, k)

        for k in range(NBLK):
            drain(k)
        build_all(rows[b])
        for k in range(NBLK):
            store_start(beta0 + k, k)

        @pl.when(c + NBUF < N_CHUNKS)
        def _():
            gather_start(c + NBUF, b)

    def outer(g, carry):
        for b in range(NBUF):
            chunk_body(g + b, b)
        return carry

    lax.fori_loop(0, N_CHUNKS // NBUF, lambda i, cr: outer(i * NBUF, cr), 0)
    for k in range(NBLK):
        beta = wid * (N_CHUNKS * NBLK) + (N_CHUNKS - 1) * NBLK + k
        store_wait(beta, k)


@functools.partial(
    pl.kernel,
    mesh=plsc.VectorSubcoreMesh(core_axis_name="c", subcore_axis_name="s"),
    out_type=jax.ShapeDtypeStruct((200, 4, 32, 8, 128), jnp.float32),
    scratch_types=[
        pltpu.VMEM((B_PER_W,), jnp.int32),
        [pltpu.VMEM((CHUNK, D), jnp.float32) for _ in range(NBUF)],
        [pltpu.VMEM((4, 8, 129), jnp.float32) for _ in range(NBLK)],
        [pltpu.SemaphoreType.DMA for _ in range(NBUF)],
        [pltpu.SemaphoreType.DMA for _ in range(NBLK)],
    ],
    compiler_params=pltpu.CompilerParams(
        use_tc_tiling_on_sc=False, needs_layout_passes=False),
)
def _gather_scale(x_hbm, table_hbm, o6_hbm, idx_v, rows, stages, gsems, ssems):
    _body(x_hbm, table_hbm, o6_hbm, idx_v, rows, stages, gsems, ssems)


_TB = 32768
_QB = _TB // 4
_NTB = -(-1000000 // _TB)


def _detile_body(tt_ref, r_ref):
    t = jnp.swapaxes(tt_ref[...], 0, 1)
    parts = [t[_QB * u:_QB * (u + 1), :] for u in range(4)]
    r_ref[...] = jnp.concatenate(parts, axis=1) * _SCALE


_detile_tc = pl.pallas_call(
    _detile_body,
    grid=(_NTB,),
    in_specs=[pl.BlockSpec((D, _TB), lambda b: (0, b))],
    out_specs=pl.BlockSpec((_QB, 128), lambda b: (b, 0)),
    out_shape=jax.ShapeDtypeStruct((_NTB * _QB, 128), jnp.float32),
)


def kernel(x, table):
    rows_lin = _detile_tc(table.T).reshape(_NTB * _TB, D)
    xt = x.T.reshape(B)
    o6 = _gather_scale(xt, rows_lin)
    return o6.transpose(2, 4, 0, 1, 3).reshape(4096, 200, 32)
